# baseline (speedup 1.0000x reference)
import functools

import jax
import jax.numpy as jnp
from jax import lax
from jax.experimental import pallas as pl
from jax.experimental.pallas import tpu as pltpu
from jax.experimental.pallas import tpu_sc as plsc

N = 10000
NP = 10240
C = 16
NS = 16
TD = 128
GD = TD
B1 = 512
BQ = 256
B4 = 512
BIG = 1e30


def _feat_body(xm_ref, xmod_ref, p_ref, einw_ref, einb_ref, qw_ref, qb_ref,
               kw_ref, kb_ref, vw_ref, vb_ref, table_ref, xq_ref, sq_ref):
    xm = xm_ref[...]
    xmod = xmod_ref[...]
    w1 = einw_ref[0:128, :]
    w2 = einw_ref[128:256, :]
    x = (jax.lax.dot_general(xm, w1, (((1,), (0,)), ((), ())),
                             preferred_element_type=jnp.float32)
         + jax.lax.dot_general(xmod, w2, (((1,), (0,)), ((), ())),
                               preferred_element_type=jnp.float32)
         + einb_ref[...])
    def lin(wref, bref):
        return jax.lax.dot_general(x, wref[...], (((1,), (0,)), ((), ())),
                                   preferred_element_type=jnp.float32) + bref[...]
    xq = lin(qw_ref, qb_ref)
    xk = lin(kw_ref, kb_ref)
    xv = lin(vw_ref, vb_ref)
    p = p_ref[...]
    pad = jnp.zeros((xm.shape[0], TD - 32 - 3), jnp.float32)
    table_ref[...] = jnp.concatenate([xk, xv, p[:, 0:3], pad], axis=1)
    xq_ref[...] = xq
    sq_ref[...] = jnp.sum(p * p, axis=1, keepdims=True)


def _features(xm_pad, xmod_pad, p_pad, E_in_W, E_in_b, q_W, q_b, k_W, k_b,
              v_W, v_b):
    nb = NP // B1
    return pl.pallas_call(
        _feat_body,
        grid=(nb,),
        in_specs=[
            pl.BlockSpec((B1, 128), lambda i: (i, 0)),
            pl.BlockSpec((B1, 128), lambda i: (i, 0)),
            pl.BlockSpec((B1, 8), lambda i: (i, 0)),
            pl.BlockSpec((256, C), lambda i: (0, 0)),
            pl.BlockSpec((1, C), lambda i: (0, 0)),
            pl.BlockSpec((C, C), lambda i: (0, 0)),
            pl.BlockSpec((1, C), lambda i: (0, 0)),
            pl.BlockSpec((C, C), lambda i: (0, 0)),
            pl.BlockSpec((1, C), lambda i: (0, 0)),
            pl.BlockSpec((C, C), lambda i: (0, 0)),
            pl.BlockSpec((1, C), lambda i: (0, 0)),
        ],
        out_specs=[
            pl.BlockSpec((B1, TD), lambda i: (i, 0)),
            pl.BlockSpec((B1, C), lambda i: (i, 0)),
            pl.BlockSpec((B1, 1), lambda i: (i, 0)),
        ],
        out_shape=[
            jax.ShapeDtypeStruct((NP, TD), jnp.float32),
            jax.ShapeDtypeStruct((NP, C), jnp.float32),
            jax.ShapeDtypeStruct((NP, 1), jnp.float32),
        ],
    )(xm_pad, xmod_pad, p_pad, E_in_W, E_in_b.reshape(1, C), q_W,
      q_b.reshape(1, C), k_W, k_b.reshape(1, C), v_W, v_b.reshape(1, C))


def _knn_body(pq_ref, pt_ref, sqc_ref, sqr_ref, idx_ref):
    cross = jax.lax.dot_general(pq_ref[...], pt_ref[...],
                                (((1,), (0,)), ((), ())),
                                preferred_element_type=jnp.float32)
    d2 = sqc_ref[...] + sqr_ref[...] - 2.0 * cross
    col = lax.broadcasted_iota(jnp.int32, d2.shape, 1)
    d2 = jnp.where(col >= N, BIG, d2)

    G = NP // 128
    v_parts = [[] for _ in range(4)]
    g_parts = [[] for _ in range(4)]
    for rg in range(BQ // 8):
        dblk = d2[rg * 8:(rg + 1) * 8, :]
        v = [jnp.full((8, 128), BIG, jnp.float32) for _ in range(4)]
        gg = [jnp.zeros((8, 128), jnp.int32) for _ in range(4)]
        for gi in range(G):
            x = dblk[:, gi * 128:(gi + 1) * 128]
            b = [x < v[k] for k in range(4)]
            m1 = jnp.maximum(v[0], x)
            m2 = jnp.maximum(v[1], m1)
            m3 = jnp.maximum(v[2], m2)
            nv = [jnp.minimum(v[0], x), jnp.minimum(v[1], m1),
                  jnp.minimum(v[2], m2), jnp.minimum(v[3], m3)]
            gic = jnp.int32(gi)
            ng = [jnp.where(b[0], gic, gg[0]),
                  jnp.where(b[0], gg[0], jnp.where(b[1], gic, gg[1])),
                  jnp.where(b[1], gg[1], jnp.where(b[2], gic, gg[2])),
                  jnp.where(b[2], gg[2], jnp.where(b[3], gic, gg[3]))]
            v, gg = nv, ng
        for k in range(4):
            v_parts[k].append(v[k])
            g_parts[k].append(gg[k])

    lane = lax.broadcasted_iota(jnp.int32, (BQ, 128), 1)
    VT = jnp.concatenate(
        [jnp.concatenate(v_parts[k], axis=0).T for k in range(4)], axis=0)
    GIT = jnp.concatenate(
        [(jnp.concatenate(g_parts[k], axis=0) * 128 + lane).T
         for k in range(4)], axis=0)
    VT0 = VT

    rows = []
    t = None
    for _ in range(NS):
        m = jnp.min(VT, axis=0, keepdims=True)
        sel = jnp.where(VT <= m, GIT, jnp.int32(2**30))
        am = jnp.min(sel, axis=0, keepdims=True)
        rows.append(am)
        VT = jnp.where(GIT == am, BIG, VT)
        t = m
    idx_fast = jnp.concatenate(rows, axis=0).T

    cnt_all = jnp.sum((d2 <= t.T).astype(jnp.int32), axis=1)
    cnt_pool = jnp.sum((VT0 <= t).astype(jnp.int32), axis=0).T
    need_fb = jnp.any(cnt_all != cnt_pool)

    def _fallback():
        dd = d2
        cols2 = []
        for _ in range(NS):
            m2 = jnp.min(dd, axis=1, keepdims=True)
            sel2 = jnp.where(dd <= m2, col, jnp.int32(2**30))
            am2 = jnp.min(sel2, axis=1, keepdims=True)
            cols2.append(am2)
            dd = jnp.where(col == am2, BIG, dd)
        return jnp.concatenate(cols2, axis=1)

    idx_ref[...] = lax.cond(need_fb, _fallback, lambda: idx_fast)


def _knn(pq_bf, pt_bf, sq_col, sq_row):
    nb = NP // BQ
    return pl.pallas_call(
        _knn_body,
        grid=(nb,),
        in_specs=[
            pl.BlockSpec((BQ, 8), lambda i: (i, 0)),
            pl.BlockSpec((8, NP), lambda i: (0, 0)),
            pl.BlockSpec((BQ, 1), lambda i: (i, 0)),
            pl.BlockSpec((1, NP), lambda i: (0, 0)),
        ],
        out_specs=pl.BlockSpec((BQ, NS), lambda i: (i, 0)),
        out_shape=jax.ShapeDtypeStruct((NP, NS), jnp.int32),
    )(pq_bf, pt_bf, sq_col, sq_row)


def _sc_gather(table, idx_flat):
    info = plsc.get_sparse_core_info()
    nw = info.num_cores * info.num_subcores
    rows_total = NP * NS
    per_w = rows_total // nw
    chunk = 256
    nchunks = per_w // chunk

    mesh = plsc.VectorSubcoreMesh(core_axis_name="c", subcore_axis_name="s")

    @functools.partial(
        pl.kernel, mesh=mesh,
        out_type=jax.ShapeDtypeStruct((rows_total, GD), jnp.float32),
        scratch_types=[
            pltpu.VMEM((chunk,), jnp.int32),
            pltpu.VMEM((chunk,), jnp.int32),
            pltpu.VMEM((chunk, TD), jnp.float32),
            pltpu.VMEM((chunk, TD), jnp.float32),
            pltpu.SemaphoreType.DMA,
            pltpu.SemaphoreType.DMA,
        ],
    )
    def k(table_hbm, idx_hbm, out_hbm, idx_a, idx_b, rows_a, rows_b,
          sem_a, sem_b):
        wid = lax.axis_index("s") * info.num_cores + lax.axis_index("c")
        base = wid * per_w
        bufs = [(idx_a, rows_a, sem_a), (idx_b, rows_b, sem_b)]
        handles = {}
        pltpu.sync_copy(idx_hbm.at[pl.ds(base, chunk)], idx_a)
        handles[0] = pltpu.async_copy(table_hbm.at[idx_a], rows_a, sem_a)
        for cix in range(nchunks):
            _, cur_rows, _ = bufs[cix % 2]
            nix = cix + 1
            if nix < nchunks:
                n_idx, n_rows, n_sem = bufs[nix % 2]
                pltpu.sync_copy(idx_hbm.at[pl.ds(base + nix * chunk, chunk)],
                                n_idx)
            handles[cix].wait()
            if nix < nchunks:
                handles[nix] = pltpu.async_copy(table_hbm.at[n_idx], n_rows,
                                                n_sem)
            pltpu.sync_copy(cur_rows, out_hbm.at[pl.ds(base + cix * chunk,
                                                       chunk)])

    return k(table, idx_flat)


def _attn_body(g_ref, xq_ref, pq_ref, xm_ref, xmod_ref, p1w_ref, p1b_ref,
               p2w_ref, p2b_ref, w1w_ref, w1b_ref, w2w_ref, w2b_ref, eow_ref,
               eob_ref, out_ref):
    g = g_ref[...]
    xk_g = g[:, 0:C]
    xv_g = g[:, C:2 * C]
    p_g = g[:, 2 * C:2 * C + 3]
    xq = jnp.broadcast_to(xq_ref[...][:, None, :],
                          (B4, NS, C)).reshape(B4 * NS, C)
    p_q = jnp.broadcast_to(pq_ref[...][:, None, 0:3],
                           (B4, NS, 3)).reshape(B4 * NS, 3)
    p_r = p_g - p_q
    pr = jax.lax.dot_general(p_r, p1w_ref[...], (((1,), (0,)), ((), ())),
                             preferred_element_type=jnp.float32) + p1b_ref[...]
    pr = jnp.maximum(pr, 0.0)
    pr = jax.lax.dot_general(pr, p2w_ref[...], (((1,), (0,)), ((), ())),
                             preferred_element_type=jnp.float32) + p2b_ref[...]
    w = jnp.maximum(xk_g - xq + pr, 0.0)
    w = jax.lax.dot_general(w, w1w_ref[...], (((1,), (0,)), ((), ())),
                            preferred_element_type=jnp.float32) + w1b_ref[...]
    w = jnp.maximum(w, 0.0)
    w = jax.lax.dot_general(w, w2w_ref[...], (((1,), (0,)), ((), ())),
                            preferred_element_type=jnp.float32) + w2b_ref[...]
    w3 = w.reshape(B4, NS, 2)
    w3 = w3 - jnp.max(w3, axis=1, keepdims=True)
    w3 = jnp.exp(w3)
    w3 = w3 / jnp.sum(w3, axis=1, keepdims=True)
    wsm = w3.reshape(B4 * NS, 2)
    wfull = jnp.concatenate([wsm] * (C // 2), axis=1)
    prod = (xv_g + pr) * wfull
    agg = jnp.sum(prod.reshape(B4, NS, C), axis=1)
    out = jax.lax.dot_general(agg, eow_ref[...], (((1,), (0,)), ((), ())),
                              preferred_element_type=jnp.float32) + eob_ref[...]
    out_ref[...] = out + jnp.concatenate([xm_ref[...], xmod_ref[...]], axis=1)


def _attention(gath, xq, p_pad, xm_pad, xmod_pad, p1_W, p1_b, p2_W, p2_b,
               w1_W, w1_b, w2_W, w2_b, E_out_W, E_out_b):
    nb = NP // B4
    return pl.pallas_call(
        _attn_body,
        grid=(nb,),
        in_specs=[
            pl.BlockSpec((B4 * NS, GD), lambda i: (i, 0)),
            pl.BlockSpec((B4, C), lambda i: (i, 0)),
            pl.BlockSpec((B4, 8), lambda i: (i, 0)),
            pl.BlockSpec((B4, 128), lambda i: (i, 0)),
            pl.BlockSpec((B4, 128), lambda i: (i, 0)),
            pl.BlockSpec((3, 3), lambda i: (0, 0)),
            pl.BlockSpec((1, 3), lambda i: (0, 0)),
            pl.BlockSpec((3, C), lambda i: (0, 0)),
            pl.BlockSpec((1, C), lambda i: (0, 0)),
            pl.BlockSpec((C, 2), lambda i: (0, 0)),
            pl.BlockSpec((1, 2), lambda i: (0, 0)),
            pl.BlockSpec((2, 2), lambda i: (0, 0)),
            pl.BlockSpec((1, 2), lambda i: (0, 0)),
            pl.BlockSpec((C, 256), lambda i: (0, 0)),
            pl.BlockSpec((1, 256), lambda i: (0, 0)),
        ],
        out_specs=pl.BlockSpec((B4, 256), lambda i: (i, 0)),
        out_shape=jax.ShapeDtypeStruct((NP, 256), jnp.float32),
    )(gath, xq, p_pad, xm_pad, xmod_pad, p1_W, p1_b.reshape(1, 3), p2_W,
      p2_b.reshape(1, C), w1_W, w1_b.reshape(1, 2), w2_W, w2_b.reshape(1, 2),
      E_out_W, E_out_b.reshape(1, 256))


def kernel(x_main, x_mod, xyz, E_in_W, E_in_b, E_out_W, E_out_b, q_W, q_b,
           k_W, k_b, v_W, v_b, p1_W, p1_b, p2_W, p2_b, w1_W, w1_b, w2_W,
           w2_b):
    pad = NP - N
    xm_pad = jnp.pad(x_main, ((0, pad), (0, 0)))
    xmod_pad = jnp.pad(x_mod, ((0, pad), (0, 0)))
    p_pad = jnp.pad(xyz.astype(jnp.float32), ((0, pad), (0, 5)))

    table, xq, sq_col = _features(xm_pad, xmod_pad, p_pad, E_in_W, E_in_b,
                                  q_W, q_b, k_W, k_b, v_W, v_b)

    pq_bf = p_pad.astype(jnp.bfloat16)
    pt_bf = pq_bf.T
    sq_row = sq_col.reshape(1, NP)
    idx = _knn(pq_bf, pt_bf, sq_col, sq_row)

    gath = _sc_gather(table, idx.reshape(NP * NS))

    out = _attention(gath, xq, p_pad, xm_pad, xmod_pad, p1_W, p1_b, p2_W,
                     p2_b, w1_W, w1_b, w2_W, w2_b, E_out_W, E_out_b)
    return out[:N]


# speedup vs baseline: 1.0210x; 1.0210x over previous
import functools

import jax
import jax.numpy as jnp
from jax import lax
from jax.experimental import pallas as pl
from jax.experimental.pallas import tpu as pltpu
from jax.experimental.pallas import tpu_sc as plsc

N = 10000
NP = 10240
C = 16
NS = 16
TD = 128
GD = TD
B1 = 512
BQ = 128
B4 = 512
BIG = 1e30


def _feat_body(xm_ref, xmod_ref, p_ref, einw_ref, einb_ref, qw_ref, qb_ref,
               kw_ref, kb_ref, vw_ref, vb_ref, table_ref, xq_ref, sq_ref):
    xm = xm_ref[...]
    xmod = xmod_ref[...]
    w1 = einw_ref[0:128, :]
    w2 = einw_ref[128:256, :]
    x = (jax.lax.dot_general(xm, w1, (((1,), (0,)), ((), ())),
                             preferred_element_type=jnp.float32)
         + jax.lax.dot_general(xmod, w2, (((1,), (0,)), ((), ())),
                               preferred_element_type=jnp.float32)
         + einb_ref[...])
    def lin(wref, bref):
        return jax.lax.dot_general(x, wref[...], (((1,), (0,)), ((), ())),
                                   preferred_element_type=jnp.float32) + bref[...]
    xq = lin(qw_ref, qb_ref)
    xk = lin(kw_ref, kb_ref)
    xv = lin(vw_ref, vb_ref)
    p = p_ref[...]
    pad = jnp.zeros((xm.shape[0], TD - 32 - 3), jnp.float32)
    table_ref[...] = jnp.concatenate([xk, xv, p[:, 0:3], pad], axis=1)
    xq_ref[...] = xq
    sq_ref[...] = jnp.sum(p * p, axis=1, keepdims=True)


def _features(xm_pad, xmod_pad, p_pad, E_in_W, E_in_b, q_W, q_b, k_W, k_b,
              v_W, v_b):
    nb = NP // B1
    return pl.pallas_call(
        _feat_body,
        grid=(nb,),
        in_specs=[
            pl.BlockSpec((B1, 128), lambda i: (i, 0)),
            pl.BlockSpec((B1, 128), lambda i: (i, 0)),
            pl.BlockSpec((B1, 8), lambda i: (i, 0)),
            pl.BlockSpec((256, C), lambda i: (0, 0)),
            pl.BlockSpec((1, C), lambda i: (0, 0)),
            pl.BlockSpec((C, C), lambda i: (0, 0)),
            pl.BlockSpec((1, C), lambda i: (0, 0)),
            pl.BlockSpec((C, C), lambda i: (0, 0)),
            pl.BlockSpec((1, C), lambda i: (0, 0)),
            pl.BlockSpec((C, C), lambda i: (0, 0)),
            pl.BlockSpec((1, C), lambda i: (0, 0)),
        ],
        out_specs=[
            pl.BlockSpec((B1, TD), lambda i: (i, 0)),
            pl.BlockSpec((B1, C), lambda i: (i, 0)),
            pl.BlockSpec((B1, 1), lambda i: (i, 0)),
        ],
        out_shape=[
            jax.ShapeDtypeStruct((NP, TD), jnp.float32),
            jax.ShapeDtypeStruct((NP, C), jnp.float32),
            jax.ShapeDtypeStruct((NP, 1), jnp.float32),
        ],
    )(xm_pad, xmod_pad, p_pad, E_in_W, E_in_b.reshape(1, C), q_W,
      q_b.reshape(1, C), k_W, k_b.reshape(1, C), v_W, v_b.reshape(1, C))


def _knn_body(pq_ref, pt_ref, sqc_ref, sqr_ref, idx_ref):
    cross = jax.lax.dot_general(pq_ref[...], pt_ref[...],
                                (((1,), (0,)), ((), ())),
                                preferred_element_type=jnp.float32)
    d2 = sqc_ref[...] + sqr_ref[...] - 2.0 * cross
    col = lax.broadcasted_iota(jnp.int32, d2.shape, 1)
    d2 = jnp.where(col >= N, BIG, d2)

    G = NP // 128
    v_parts = [[] for _ in range(4)]
    g_parts = [[] for _ in range(4)]
    for rg in range(BQ // 8):
        dblk = d2[rg * 8:(rg + 1) * 8, :]
        v = [jnp.full((8, 128), BIG, jnp.float32) for _ in range(4)]
        gg = [jnp.zeros((8, 128), jnp.int32) for _ in range(4)]
        for gi in range(G):
            x = dblk[:, gi * 128:(gi + 1) * 128]
            b = [x < v[k] for k in range(4)]
            m1 = jnp.maximum(v[0], x)
            m2 = jnp.maximum(v[1], m1)
            m3 = jnp.maximum(v[2], m2)
            nv = [jnp.minimum(v[0], x), jnp.minimum(v[1], m1),
                  jnp.minimum(v[2], m2), jnp.minimum(v[3], m3)]
            gic = jnp.int32(gi)
            ng = [jnp.where(b[0], gic, gg[0]),
                  jnp.where(b[0], gg[0], jnp.where(b[1], gic, gg[1])),
                  jnp.where(b[1], gg[1], jnp.where(b[2], gic, gg[2])),
                  jnp.where(b[2], gg[2], jnp.where(b[3], gic, gg[3]))]
            v, gg = nv, ng
        for k in range(4):
            v_parts[k].append(v[k])
            g_parts[k].append(gg[k])

    lane = lax.broadcasted_iota(jnp.int32, (BQ, 128), 1)
    VT = jnp.concatenate(
        [jnp.concatenate(v_parts[k], axis=0).T for k in range(4)], axis=0)
    GIT = jnp.concatenate(
        [(jnp.concatenate(g_parts[k], axis=0) * 128 + lane).T
         for k in range(4)], axis=0)
    VT0 = VT

    rows = []
    t = None
    for _ in range(NS):
        m = jnp.min(VT, axis=0, keepdims=True)
        sel = jnp.where(VT <= m, GIT, jnp.int32(2**30))
        am = jnp.min(sel, axis=0, keepdims=True)
        rows.append(am)
        VT = jnp.where(GIT == am, BIG, VT)
        t = m
    idx_fast = jnp.concatenate(rows, axis=0).T

    cnt_all = jnp.sum((d2 <= t.T).astype(jnp.int32), axis=1)
    cnt_pool = jnp.sum((VT0 <= t).astype(jnp.int32), axis=0).T
    need_fb = jnp.any(cnt_all != cnt_pool)

    def _fallback():
        dd = d2
        cols2 = []
        for _ in range(NS):
            m2 = jnp.min(dd, axis=1, keepdims=True)
            sel2 = jnp.where(dd <= m2, col, jnp.int32(2**30))
            am2 = jnp.min(sel2, axis=1, keepdims=True)
            cols2.append(am2)
            dd = jnp.where(col == am2, BIG, dd)
        return jnp.concatenate(cols2, axis=1)

    idx_ref[...] = lax.cond(need_fb, _fallback, lambda: idx_fast)


def _knn(pq_bf, pt_bf, sq_col, sq_row):
    nb = NP // BQ
    return pl.pallas_call(
        _knn_body,
        grid=(nb,),
        in_specs=[
            pl.BlockSpec((BQ, 8), lambda i: (i, 0)),
            pl.BlockSpec((8, NP), lambda i: (0, 0)),
            pl.BlockSpec((BQ, 1), lambda i: (i, 0)),
            pl.BlockSpec((1, NP), lambda i: (0, 0)),
        ],
        out_specs=pl.BlockSpec((BQ, NS), lambda i: (i, 0)),
        out_shape=jax.ShapeDtypeStruct((NP, NS), jnp.int32),
    )(pq_bf, pt_bf, sq_col, sq_row)


def _sc_gather(table, idx_flat):
    info = plsc.get_sparse_core_info()
    nw = info.num_cores * info.num_subcores
    rows_total = NP * NS
    per_w = rows_total // nw
    chunk = 256
    nchunks = per_w // chunk

    mesh = plsc.VectorSubcoreMesh(core_axis_name="c", subcore_axis_name="s")

    @functools.partial(
        pl.kernel, mesh=mesh,
        out_type=jax.ShapeDtypeStruct((rows_total, GD), jnp.float32),
        scratch_types=[
            pltpu.VMEM((chunk,), jnp.int32),
            pltpu.VMEM((chunk,), jnp.int32),
            pltpu.VMEM((chunk, TD), jnp.float32),
            pltpu.VMEM((chunk, TD), jnp.float32),
            pltpu.SemaphoreType.DMA,
            pltpu.SemaphoreType.DMA,
        ],
    )
    def k(table_hbm, idx_hbm, out_hbm, idx_a, idx_b, rows_a, rows_b,
          sem_a, sem_b):
        wid = lax.axis_index("s") * info.num_cores + lax.axis_index("c")
        base = wid * per_w
        bufs = [(idx_a, rows_a, sem_a), (idx_b, rows_b, sem_b)]
        handles = {}
        pltpu.sync_copy(idx_hbm.at[pl.ds(base, chunk)], idx_a)
        handles[0] = pltpu.async_copy(table_hbm.at[idx_a], rows_a, sem_a)
        for cix in range(nchunks):
            _, cur_rows, _ = bufs[cix % 2]
            nix = cix + 1
            if nix < nchunks:
                n_idx, n_rows, n_sem = bufs[nix % 2]
                pltpu.sync_copy(idx_hbm.at[pl.ds(base + nix * chunk, chunk)],
                                n_idx)
            handles[cix].wait()
            if nix < nchunks:
                handles[nix] = pltpu.async_copy(table_hbm.at[n_idx], n_rows,
                                                n_sem)
            pltpu.sync_copy(cur_rows, out_hbm.at[pl.ds(base + cix * chunk,
                                                       chunk)])

    return k(table, idx_flat)


def _attn_body(g_ref, xq_ref, pq_ref, xm_ref, xmod_ref, p1w_ref, p1b_ref,
               p2w_ref, p2b_ref, w1w_ref, w1b_ref, w2w_ref, w2b_ref, eow_ref,
               eob_ref, out_ref):
    g = g_ref[...]
    xk_g = g[:, 0:C]
    xv_g = g[:, C:2 * C]
    p_g = g[:, 2 * C:2 * C + 3]
    xq = jnp.broadcast_to(xq_ref[...][:, None, :],
                          (B4, NS, C)).reshape(B4 * NS, C)
    p_q = jnp.broadcast_to(pq_ref[...][:, None, 0:3],
                           (B4, NS, 3)).reshape(B4 * NS, 3)
    p_r = p_g - p_q
    pr = jax.lax.dot_general(p_r, p1w_ref[...], (((1,), (0,)), ((), ())),
                             preferred_element_type=jnp.float32) + p1b_ref[...]
    pr = jnp.maximum(pr, 0.0)
    pr = jax.lax.dot_general(pr, p2w_ref[...], (((1,), (0,)), ((), ())),
                             preferred_element_type=jnp.float32) + p2b_ref[...]
    w = jnp.maximum(xk_g - xq + pr, 0.0)
    w = jax.lax.dot_general(w, w1w_ref[...], (((1,), (0,)), ((), ())),
                            preferred_element_type=jnp.float32) + w1b_ref[...]
    w = jnp.maximum(w, 0.0)
    w = jax.lax.dot_general(w, w2w_ref[...], (((1,), (0,)), ((), ())),
                            preferred_element_type=jnp.float32) + w2b_ref[...]
    w3 = w.reshape(B4, NS, 2)
    w3 = w3 - jnp.max(w3, axis=1, keepdims=True)
    w3 = jnp.exp(w3)
    w3 = w3 / jnp.sum(w3, axis=1, keepdims=True)
    wsm = w3.reshape(B4 * NS, 2)
    wfull = jnp.concatenate([wsm] * (C // 2), axis=1)
    prod = (xv_g + pr) * wfull
    agg = jnp.sum(prod.reshape(B4, NS, C), axis=1)
    out = jax.lax.dot_general(agg, eow_ref[...], (((1,), (0,)), ((), ())),
                              preferred_element_type=jnp.float32) + eob_ref[...]
    out_ref[...] = out + jnp.concatenate([xm_ref[...], xmod_ref[...]], axis=1)


def _attention(gath, xq, p_pad, xm_pad, xmod_pad, p1_W, p1_b, p2_W, p2_b,
               w1_W, w1_b, w2_W, w2_b, E_out_W, E_out_b):
    nb = NP // B4
    return pl.pallas_call(
        _attn_body,
        grid=(nb,),
        in_specs=[
            pl.BlockSpec((B4 * NS, GD), lambda i: (i, 0)),
            pl.BlockSpec((B4, C), lambda i: (i, 0)),
            pl.BlockSpec((B4, 8), lambda i: (i, 0)),
            pl.BlockSpec((B4, 128), lambda i: (i, 0)),
            pl.BlockSpec((B4, 128), lambda i: (i, 0)),
            pl.BlockSpec((3, 3), lambda i: (0, 0)),
            pl.BlockSpec((1, 3), lambda i: (0, 0)),
            pl.BlockSpec((3, C), lambda i: (0, 0)),
            pl.BlockSpec((1, C), lambda i: (0, 0)),
            pl.BlockSpec((C, 2), lambda i: (0, 0)),
            pl.BlockSpec((1, 2), lambda i: (0, 0)),
            pl.BlockSpec((2, 2), lambda i: (0, 0)),
            pl.BlockSpec((1, 2), lambda i: (0, 0)),
            pl.BlockSpec((C, 256), lambda i: (0, 0)),
            pl.BlockSpec((1, 256), lambda i: (0, 0)),
        ],
        out_specs=pl.BlockSpec((B4, 256), lambda i: (i, 0)),
        out_shape=jax.ShapeDtypeStruct((NP, 256), jnp.float32),
    )(gath, xq, p_pad, xm_pad, xmod_pad, p1_W, p1_b.reshape(1, 3), p2_W,
      p2_b.reshape(1, C), w1_W, w1_b.reshape(1, 2), w2_W, w2_b.reshape(1, 2),
      E_out_W, E_out_b.reshape(1, 256))


def kernel(x_main, x_mod, xyz, E_in_W, E_in_b, E_out_W, E_out_b, q_W, q_b,
           k_W, k_b, v_W, v_b, p1_W, p1_b, p2_W, p2_b, w1_W, w1_b, w2_W,
           w2_b):
    pad = NP - N
    xm_pad = jnp.pad(x_main, ((0, pad), (0, 0)))
    xmod_pad = jnp.pad(x_mod, ((0, pad), (0, 0)))
    p_pad = jnp.pad(xyz.astype(jnp.float32), ((0, pad), (0, 5)))

    table, xq, sq_col = _features(xm_pad, xmod_pad, p_pad, E_in_W, E_in_b,
                                  q_W, q_b, k_W, k_b, v_W, v_b)

    pq_bf = p_pad.astype(jnp.bfloat16)
    pt_bf = pq_bf.T
    sq_row = sq_col.reshape(1, NP)
    idx = _knn(pq_bf, pt_bf, sq_col, sq_row)

    gath = _sc_gather(table, idx.reshape(NP * NS))

    out = _attention(gath, xq, p_pad, xm_pad, xmod_pad, p1_W, p1_b, p2_W,
                     p2_b, w1_W, w1_b, w2_W, w2_b, E_out_W, E_out_b)
    return out[:N]
